# baseline (speedup 1.0000x reference)
import functools

import jax
import jax.numpy as jnp
from jax.experimental import pallas as pl
from jax.experimental.pallas import tpu as pltpu

_PARALLEL = pltpu.CompilerParams(dimension_semantics=("parallel",))

NUM_TIMES = 64
EMBED_DIM = 64
TOP_K = 10
TIME_WINDOW = 8

_ROW_BLK = 256
_BATCH_BLK = 512


def _topk_nei_kernel(sim_ref, emb_ref, w_ref, out_ref):
    vals = sim_ref[...]
    r, n = vals.shape
    col = jax.lax.broadcasted_iota(jnp.int32, (r, n), 1)
    acc = jnp.zeros((r, n), jnp.float32)
    for k in range(TOP_K):
        m = jnp.max(vals, axis=1, keepdims=True)
        idx = jnp.min(jnp.where(vals == m, col, n), axis=1, keepdims=True)
        hit = col == idx
        acc = acc + jnp.where(hit, w_ref[0, k], 0.0)
        vals = jnp.where(hit, -jnp.inf, vals)
    nei = jnp.dot(acc, emb_ref[...], preferred_element_type=jnp.float32,
                precision=jax.lax.Precision.HIGHEST)
    out_ref[...] = nei + w_ref[0, TOP_K]


def _neighbor_combine(sim, emb, cnn_w, cnn_b):
    n = sim.shape[0]
    wpack = jnp.zeros((1, 128), jnp.float32)
    wpack = wpack.at[0, :TOP_K].set(cnn_w)
    wpack = wpack.at[0, TOP_K].set(cnn_b[0])
    return pl.pallas_call(
        _topk_nei_kernel,
        grid=(n // _ROW_BLK,),
        in_specs=[
            pl.BlockSpec((_ROW_BLK, n), lambda b: (b, 0)),
            pl.BlockSpec((n, EMBED_DIM), lambda b: (0, 0)),
            pl.BlockSpec((1, 128), lambda b: (0, 0)),
        ],
        out_specs=pl.BlockSpec((_ROW_BLK, EMBED_DIM), lambda b: (b, 0)),
        out_shape=jax.ShapeDtypeStruct((n, EMBED_DIM), jnp.float32),
        compiler_params=_PARALLEL,
    )(sim, emb, wpack)


def _batch_kernel(ui_ref, gu_ref, gi_ref, hist_ref,
                  intW_ref, fgW_ref, Wzh_ref, Wrh_ref, Whh_ref, Wch_ref,
                  fcW_ref, wx_ref, sp_ref, out_ref):
    u_col = ui_ref[:, 0:1]
    i_col = ui_ref[:, 1:2]
    b = u_col.shape[0]
    n = gu_ref.shape[0]
    col = jax.lax.broadcasted_iota(jnp.int32, (b, n), 1)
    su = (col == u_col).astype(jnp.float32)
    uu = jnp.dot(su, gu_ref[...], preferred_element_type=jnp.float32,
                 precision=jax.lax.Precision.HIGHEST)
    si = (col == i_col).astype(jnp.float32)
    vi = jnp.dot(si, gi_ref[...], preferred_element_type=jnp.float32,
                 precision=jax.lax.Precision.HIGHEST)
    u_emb, u_nei = uu[:, :EMBED_DIM], uu[:, EMBED_DIM:]
    i_emb, i_nei = vi[:, :EMBED_DIM], vi[:, EMBED_DIM:]

    int_b = sp_ref[0:1, :]
    fg_b = sp_ref[1:2, :]
    b_z = sp_ref[2:3, :]
    b_r = sp_ref[3:4, :]
    b_h = sp_ref[4:5, :]
    b_c = sp_ref[5:6, :]
    fc_b = sp_ref[6:7, :]
    pred_w = sp_ref[7:8, :]
    w_zx = wx_ref[0:1, :]
    w_rx = wx_ref[1:2, :]
    w_hx = wx_ref[2:3, :]
    pred_b = wx_ref[3:4, 0:1]

    int_w = intW_ref[...]
    fg_w = fgW_ref[...]

    def interaction(a, bb):
        xv = a + bb
        fr = jax.nn.sigmoid(
            jnp.dot(xv, fg_w, preferred_element_type=jnp.float32,
                precision=jax.lax.Precision.HIGHEST) + fg_b)
        t = jnp.dot(xv, int_w, preferred_element_type=jnp.float32,
                precision=jax.lax.Precision.HIGHEST) + int_b
        return fr * t + (1.0 - fr) * xv

    cross = jnp.concatenate(
        [interaction(u_emb, u_nei), interaction(i_emb, i_nei), u_emb, i_emb],
        axis=1)
    state = jax.nn.relu(
        jnp.dot(cross, fcW_ref[...], preferred_element_type=jnp.float32,
                precision=jax.lax.Precision.HIGHEST) + fc_b)

    w_zh = Wzh_ref[...]
    w_rh = Wrh_ref[...]
    w_hh = Whh_ref[...]
    h = state
    c = jax.nn.relu(
        jnp.dot(state, Wch_ref[...], preferred_element_type=jnp.float32,
                precision=jax.lax.Precision.HIGHEST) + b_c)
    for t in range(TIME_WINDOW):
        xt = hist_ref[:, t:t + 1]
        z = jax.nn.sigmoid(
            jnp.dot(h, w_zh, preferred_element_type=jnp.float32,
                precision=jax.lax.Precision.HIGHEST)
            + xt * w_zx + b_z + c)
        rr = jax.nn.sigmoid(
            jnp.dot(h, w_rh, preferred_element_type=jnp.float32,
                precision=jax.lax.Precision.HIGHEST)
            + xt * w_rx + b_r + c)
        ht = jnp.tanh(
            jnp.dot(h * rr, w_hh, preferred_element_type=jnp.float32,
                precision=jax.lax.Precision.HIGHEST)
            + xt * w_hx + b_h + c)
        h = (1.0 - z) * h + z * ht
    y = jnp.sum(h * pred_w, axis=1, keepdims=True) + pred_b
    out_ref[...] = jnp.broadcast_to(y, (b, 128))


def kernel(x, uu_sim, ii_sim, hist, user_embedding, item_embedding,
           user_cnn_w, user_cnn_b, item_cnn_w, item_cnn_b,
           int_W, int_b, fg_W, fg_b,
           W_zh, W_zx, b_z, W_rh, W_rx, b_r, W_hh, W_hx, b_h, W_ch, b_c,
           fc_W, fc_b, pred_W, pred_b):
    batch = x.shape[0]
    n_u = uu_sim.shape[0]
    n_i = ii_sim.shape[0]
    d = EMBED_DIM

    nei_u = _neighbor_combine(uu_sim, user_embedding, user_cnn_w, user_cnn_b)
    nei_i = _neighbor_combine(ii_sim, item_embedding, item_cnn_w, item_cnn_b)

    g_u = jnp.concatenate([user_embedding, nei_u], axis=1)
    g_i = jnp.concatenate([item_embedding, nei_i], axis=1)

    ui = jnp.pad(x[:, 1:3], ((0, 0), (0, 126)))
    hist_t = hist.T

    wx = jnp.zeros((4, d), jnp.float32)
    wx = wx.at[0, :].set(W_zx[0])
    wx = wx.at[1, :].set(W_rx[0])
    wx = wx.at[2, :].set(W_hx[0])
    wx = wx.at[3, 0].set(pred_b[0])
    sp = jnp.stack([int_b, fg_b, b_z, b_r, b_h, b_c, fc_b, pred_W[:, 0]])

    full = lambda shape: pl.BlockSpec(shape, lambda bb: (0, 0))
    out = pl.pallas_call(
        _batch_kernel,
        grid=(batch // _BATCH_BLK,),
        in_specs=[
            pl.BlockSpec((_BATCH_BLK, 128), lambda bb: (bb, 0)),
            full((n_u, 2 * d)),
            full((n_i, 2 * d)),
            pl.BlockSpec((_BATCH_BLK, TIME_WINDOW), lambda bb: (bb, 0)),
            full((d, d)), full((d, d)), full((d, d)), full((d, d)),
            full((d, d)), full((d, d)), full((4 * d, d)),
            full((4, d)), full((8, d)),
        ],
        out_specs=pl.BlockSpec((_BATCH_BLK, 128), lambda bb: (bb, 0)),
        out_shape=jax.ShapeDtypeStruct((batch, 128), jnp.float32),
        compiler_params=_PARALLEL,
    )(ui, g_u, g_i, hist_t, int_W, fg_W, W_zh, W_rh, W_hh, W_ch, fc_W, wx, sp)
    return out[:, 0]


# speedup vs baseline: 1.1544x; 1.1544x over previous
import functools

import jax
import jax.numpy as jnp
from jax import lax
from jax.experimental import pallas as pl
from jax.experimental.pallas import tpu as pltpu
from jax.experimental.pallas import tpu_sc as plsc

_PARALLEL = pltpu.CompilerParams(dimension_semantics=("parallel",))

_SC_NC = 2
_SC_NW = 32


def _sc_gather_pair(g_u, g_i, u_idx, i_idx):
    n, dd = g_u.shape
    b = u_idx.shape[0]
    bpw = b // _SC_NW
    mesh = plsc.VectorSubcoreMesh(core_axis_name="c", subcore_axis_name="s")

    @functools.partial(
        pl.kernel,
        out_type=(jax.ShapeDtypeStruct((b, dd), jnp.float32),
                  jax.ShapeDtypeStruct((b, dd), jnp.float32)),
        mesh=mesh,
        scratch_types=[
            pltpu.VMEM((bpw,), jnp.int32),
            pltpu.VMEM((bpw, dd), jnp.float32),
            pltpu.SemaphoreType.DMA,
        ],
    )
    def gather_kernel(gu_hbm, gi_hbm, u_hbm, i_hbm, out_u_hbm, out_i_hbm,
                      idx_v, rows_v, sem):
        wid = lax.axis_index("s") * _SC_NC + lax.axis_index("c")
        base = wid * bpw
        pltpu.sync_copy(u_hbm.at[pl.ds(base, bpw)], idx_v)
        pltpu.async_copy(gu_hbm.at[idx_v], rows_v, sem).wait()
        pltpu.sync_copy(rows_v, out_u_hbm.at[pl.ds(base, bpw)])
        pltpu.sync_copy(i_hbm.at[pl.ds(base, bpw)], idx_v)
        pltpu.async_copy(gi_hbm.at[idx_v], rows_v, sem).wait()
        pltpu.sync_copy(rows_v, out_i_hbm.at[pl.ds(base, bpw)])

    return gather_kernel(g_u, g_i, u_idx, i_idx)

NUM_TIMES = 64
EMBED_DIM = 64
TOP_K = 10
TIME_WINDOW = 8

_ROW_BLK = 256
_BATCH_BLK = 512


def _topk_nei_kernel(sim_ref, emb_ref, w_ref, out_ref):
    vals = sim_ref[...]
    r, n = vals.shape
    col = jax.lax.broadcasted_iota(jnp.int32, (r, n), 1)
    acc = jnp.zeros((r, n), jnp.float32)
    for k in range(TOP_K):
        m = jnp.max(vals, axis=1, keepdims=True)
        idx = jnp.min(jnp.where(vals == m, col, n), axis=1, keepdims=True)
        hit = col == idx
        acc = acc + jnp.where(hit, w_ref[0, k], 0.0)
        vals = jnp.where(hit, -jnp.inf, vals)
    nei = jnp.dot(acc, emb_ref[...], preferred_element_type=jnp.float32,
                precision=jax.lax.Precision.HIGHEST)
    out_ref[...] = nei + w_ref[0, TOP_K]


def _neighbor_combine(sim, emb, cnn_w, cnn_b):
    n = sim.shape[0]
    wpack = jnp.zeros((1, 128), jnp.float32)
    wpack = wpack.at[0, :TOP_K].set(cnn_w)
    wpack = wpack.at[0, TOP_K].set(cnn_b[0])
    return pl.pallas_call(
        _topk_nei_kernel,
        grid=(n // _ROW_BLK,),
        in_specs=[
            pl.BlockSpec((_ROW_BLK, n), lambda b: (b, 0)),
            pl.BlockSpec((n, EMBED_DIM), lambda b: (0, 0)),
            pl.BlockSpec((1, 128), lambda b: (0, 0)),
        ],
        out_specs=pl.BlockSpec((_ROW_BLK, EMBED_DIM), lambda b: (b, 0)),
        out_shape=jax.ShapeDtypeStruct((n, EMBED_DIM), jnp.float32),
        compiler_params=_PARALLEL,
    )(sim, emb, wpack)


def _batch_kernel(gu_ref, gi_ref, hist_ref,
                  intW_ref, fgW_ref, Wzh_ref, Wrh_ref, Whh_ref, Wch_ref,
                  fcW_ref, wx_ref, sp_ref, out_ref):
    uu = gu_ref[...]
    vi = gi_ref[...]
    b = uu.shape[0]
    u_emb, u_nei = uu[:, :EMBED_DIM], uu[:, EMBED_DIM:]
    i_emb, i_nei = vi[:, :EMBED_DIM], vi[:, EMBED_DIM:]

    int_b = sp_ref[0:1, :]
    fg_b = sp_ref[1:2, :]
    b_z = sp_ref[2:3, :]
    b_r = sp_ref[3:4, :]
    b_h = sp_ref[4:5, :]
    b_c = sp_ref[5:6, :]
    fc_b = sp_ref[6:7, :]
    pred_w = sp_ref[7:8, :]
    w_zx = wx_ref[0:1, :]
    w_rx = wx_ref[1:2, :]
    w_hx = wx_ref[2:3, :]
    pred_b = wx_ref[3:4, 0:1]

    int_w = intW_ref[...]
    fg_w = fgW_ref[...]

    def interaction(a, bb):
        xv = a + bb
        fr = jax.nn.sigmoid(
            jnp.dot(xv, fg_w, preferred_element_type=jnp.float32,
                precision=jax.lax.Precision.HIGHEST) + fg_b)
        t = jnp.dot(xv, int_w, preferred_element_type=jnp.float32,
                precision=jax.lax.Precision.HIGHEST) + int_b
        return fr * t + (1.0 - fr) * xv

    cross = jnp.concatenate(
        [interaction(u_emb, u_nei), interaction(i_emb, i_nei), u_emb, i_emb],
        axis=1)
    state = jax.nn.relu(
        jnp.dot(cross, fcW_ref[...], preferred_element_type=jnp.float32,
                precision=jax.lax.Precision.HIGHEST) + fc_b)

    w_zh = Wzh_ref[...]
    w_rh = Wrh_ref[...]
    w_hh = Whh_ref[...]
    h = state
    c = jax.nn.relu(
        jnp.dot(state, Wch_ref[...], preferred_element_type=jnp.float32,
                precision=jax.lax.Precision.HIGHEST) + b_c)
    for t in range(TIME_WINDOW):
        xt = hist_ref[:, t:t + 1]
        z = jax.nn.sigmoid(
            jnp.dot(h, w_zh, preferred_element_type=jnp.float32,
                precision=jax.lax.Precision.HIGHEST)
            + xt * w_zx + b_z + c)
        rr = jax.nn.sigmoid(
            jnp.dot(h, w_rh, preferred_element_type=jnp.float32,
                precision=jax.lax.Precision.HIGHEST)
            + xt * w_rx + b_r + c)
        ht = jnp.tanh(
            jnp.dot(h * rr, w_hh, preferred_element_type=jnp.float32,
                precision=jax.lax.Precision.HIGHEST)
            + xt * w_hx + b_h + c)
        h = (1.0 - z) * h + z * ht
    y = jnp.sum(h * pred_w, axis=1, keepdims=True) + pred_b
    out_ref[...] = jnp.broadcast_to(y, (b, 128))


def kernel(x, uu_sim, ii_sim, hist, user_embedding, item_embedding,
           user_cnn_w, user_cnn_b, item_cnn_w, item_cnn_b,
           int_W, int_b, fg_W, fg_b,
           W_zh, W_zx, b_z, W_rh, W_rx, b_r, W_hh, W_hx, b_h, W_ch, b_c,
           fc_W, fc_b, pred_W, pred_b):
    batch = x.shape[0]
    n_u = uu_sim.shape[0]
    n_i = ii_sim.shape[0]
    d = EMBED_DIM

    nei_u = _neighbor_combine(uu_sim, user_embedding, user_cnn_w, user_cnn_b)
    nei_i = _neighbor_combine(ii_sim, item_embedding, item_cnn_w, item_cnn_b)

    g_u = jnp.concatenate([user_embedding, nei_u], axis=1)
    g_i = jnp.concatenate([item_embedding, nei_i], axis=1)

    u_idx = x[:, 1]
    i_idx = x[:, 2]
    rows_u, rows_i = _sc_gather_pair(g_u, g_i, u_idx, i_idx)
    hist_t = hist.T

    wx = jnp.zeros((4, d), jnp.float32)
    wx = wx.at[0, :].set(W_zx[0])
    wx = wx.at[1, :].set(W_rx[0])
    wx = wx.at[2, :].set(W_hx[0])
    wx = wx.at[3, 0].set(pred_b[0])
    sp = jnp.stack([int_b, fg_b, b_z, b_r, b_h, b_c, fc_b, pred_W[:, 0]])

    full = lambda shape: pl.BlockSpec(shape, lambda bb: (0, 0))
    out = pl.pallas_call(
        _batch_kernel,
        grid=(batch // _BATCH_BLK,),
        in_specs=[
            pl.BlockSpec((_BATCH_BLK, 2 * d), lambda bb: (bb, 0)),
            pl.BlockSpec((_BATCH_BLK, 2 * d), lambda bb: (bb, 0)),
            pl.BlockSpec((_BATCH_BLK, TIME_WINDOW), lambda bb: (bb, 0)),
            full((d, d)), full((d, d)), full((d, d)), full((d, d)),
            full((d, d)), full((d, d)), full((4 * d, d)),
            full((4, d)), full((8, d)),
        ],
        out_specs=pl.BlockSpec((_BATCH_BLK, 128), lambda bb: (bb, 0)),
        out_shape=jax.ShapeDtypeStruct((batch, 128), jnp.float32),
        compiler_params=_PARALLEL,
    )(rows_u, rows_i, hist_t, int_W, fg_W, W_zh, W_rh, W_hh, W_ch, fc_W, wx, sp)
    return out[:, 0]


# speedup vs baseline: 1.6079x; 1.3928x over previous
import functools

import jax
import jax.numpy as jnp
from jax import lax
from jax.experimental import pallas as pl
from jax.experimental.pallas import tpu as pltpu
from jax.experimental.pallas import tpu_sc as plsc

NUM_TIMES = 64
EMBED_DIM = 64
TOP_K = 10
TIME_WINDOW = 8

_ROW_BLK = 256
_BATCH_BLK = 512

_PARALLEL = pltpu.CompilerParams(dimension_semantics=("parallel",))

_SC_NC = 2
_SC_NW = 32
_LANES = 16


def _topk_idx_kernel(sim_ref, out_ref):
    vals = sim_ref[...]
    r, n = vals.shape
    col = lax.broadcasted_iota(jnp.int32, (r, n), 1)
    idxs = []
    for _ in range(TOP_K):
        m = jnp.max(vals, axis=1, keepdims=True)
        idx = jnp.min(jnp.where(vals == m, col, n), axis=1, keepdims=True)
        idxs.append(idx)
        vals = jnp.where(col == idx, -jnp.inf, vals)
    pad = jnp.broadcast_to(idxs[0], (r, 128 - TOP_K))
    out_ref[...] = jnp.concatenate(idxs + [pad], axis=1)


def _topk_indices(sim):
    n = sim.shape[0]
    return pl.pallas_call(
        _topk_idx_kernel,
        grid=(n // _ROW_BLK,),
        in_specs=[pl.BlockSpec((_ROW_BLK, n), lambda b: (b, 0))],
        out_specs=pl.BlockSpec((_ROW_BLK, 128), lambda b: (b, 0)),
        out_shape=jax.ShapeDtypeStruct((n, 128), jnp.int32),
        compiler_params=_PARALLEL,
    )(sim)


def _sc_combine(emb_u, tk_u, u_idx, w_u, emb_i, tk_i, i_idx, w_i):
    b = u_idx.shape[0]
    dd = emb_u.shape[1]
    bpw = b // _SC_NW
    sub = bpw // 2
    nj = EMBED_DIM // _LANES
    mesh = plsc.VectorSubcoreMesh(core_axis_name="c", subcore_axis_name="s")
    out_sd = jax.ShapeDtypeStruct((b, dd), jnp.float32)

    @functools.partial(
        pl.kernel,
        out_type=(out_sd, out_sd, out_sd, out_sd),
        mesh=mesh,
        compiler_params=pltpu.CompilerParams(needs_layout_passes=False),
        scratch_types=[
            pltpu.VMEM((sub,), jnp.int32),
            pltpu.VMEM((sub, 128), jnp.int32),
            pltpu.VMEM((16, sub), jnp.int32),
            pltpu.VMEM((sub, dd), jnp.float32),
            pltpu.VMEM((TOP_K, sub, dd), jnp.float32),
            pltpu.VMEM((sub, dd), jnp.float32),
            pltpu.VMEM((16, 16), jnp.float32),
            pltpu.SemaphoreType.DMA,
        ],
    )
    def combine_kernel(embu_hbm, tku_hbm, uid_hbm, wu_hbm,
                       embi_hbm, tki_hbm, iid_hbm, wi_hbm,
                       oselfu_hbm, oneiu_hbm, oselfi_hbm, oneii_hbm,
                       ids_v, tk_v, idxk_v, self_v, rows_v, nei_v, w_v, sem):
        wid = lax.axis_index("s") * _SC_NC + lax.axis_index("c")
        zero = jnp.zeros((_LANES,), jnp.float32)
        lane = lax.iota(jnp.int32, _LANES)

        def one_chunk(emb_hbm, tk_hbm, id_hbm, oself_hbm, onei_hbm, base):
            pltpu.sync_copy(id_hbm.at[pl.ds(base, sub)], ids_v)
            pltpu.async_copy(tk_hbm.at[ids_v], tk_v, sem).wait()
            pltpu.async_copy(emb_hbm.at[ids_v], self_v, sem).wait()
            pltpu.sync_copy(self_v, oself_hbm.at[pl.ds(base, sub)])
            copies = []
            for k in range(TOP_K):
                kcol = jnp.full((_LANES,), k, jnp.int32)
                for c in range(sub // _LANES):
                    rows16 = plsc.load_gather(
                        tk_v, [lane + _LANES * c, kcol])
                    idxk_v[k, pl.ds(_LANES * c, _LANES)] = rows16
                copies.append(
                    pltpu.async_copy(emb_hbm.at[idxk_v.at[k]],
                                     rows_v.at[k], sem))
            for cp in copies:
                cp.wait()

            def body(bb, _):
                for j in range(nj):
                    acc = zero
                    for k in range(TOP_K):
                        acc = acc + w_v[k] * rows_v[k, bb,
                                                    pl.ds(_LANES * j, _LANES)]
                    nei_v[bb, pl.ds(_LANES * j, _LANES)] = acc
                for j in range(nj, dd // _LANES):
                    nei_v[bb, pl.ds(_LANES * j, _LANES)] = zero
                return 0

            lax.fori_loop(0, sub, body, 0)
            pltpu.sync_copy(nei_v, onei_hbm.at[pl.ds(base, sub)])

        pltpu.sync_copy(wu_hbm, w_v)
        for h in range(2):
            one_chunk(embu_hbm, tku_hbm, uid_hbm, oselfu_hbm, oneiu_hbm,
                      wid * bpw + h * sub)
        pltpu.sync_copy(wi_hbm, w_v)
        for h in range(2):
            one_chunk(embi_hbm, tki_hbm, iid_hbm, oselfi_hbm, oneii_hbm,
                      wid * bpw + h * sub)

    return combine_kernel(emb_u, tk_u, u_idx, w_u, emb_i, tk_i, i_idx, w_i)


def _batch_kernel(ue_ref, un_ref, ie_ref, in_ref, hist_ref,
                  intW_ref, fgW_ref, Wzh_ref, Wrh_ref, Whh_ref, Wch_ref,
                  fcW_ref, wx_ref, sp_ref, out_ref):
    u_emb = ue_ref[:, :EMBED_DIM]
    i_emb = ie_ref[:, :EMBED_DIM]
    u_nei = un_ref[:, :EMBED_DIM] + wx_ref[3:4, 1:2]
    i_nei = in_ref[:, :EMBED_DIM] + wx_ref[3:4, 2:3]
    b = u_emb.shape[0]

    int_b = sp_ref[0:1, :]
    fg_b = sp_ref[1:2, :]
    b_z = sp_ref[2:3, :]
    b_r = sp_ref[3:4, :]
    b_h = sp_ref[4:5, :]
    b_c = sp_ref[5:6, :]
    fc_b = sp_ref[6:7, :]
    pred_w = sp_ref[7:8, :]
    w_zx = wx_ref[0:1, :]
    w_rx = wx_ref[1:2, :]
    w_hx = wx_ref[2:3, :]
    pred_b = wx_ref[3:4, 0:1]

    int_w = intW_ref[...]
    fg_w = fgW_ref[...]

    def interaction(a, bb):
        xv = a + bb
        fr = jax.nn.sigmoid(
            jnp.dot(xv, fg_w, preferred_element_type=jnp.float32,
                    precision=jax.lax.Precision.HIGHEST) + fg_b)
        t = jnp.dot(xv, int_w, preferred_element_type=jnp.float32,
                    precision=jax.lax.Precision.HIGHEST) + int_b
        return fr * t + (1.0 - fr) * xv

    cross = jnp.concatenate(
        [interaction(u_emb, u_nei), interaction(i_emb, i_nei), u_emb, i_emb],
        axis=1)
    state = jax.nn.relu(
        jnp.dot(cross, fcW_ref[...], preferred_element_type=jnp.float32,
                precision=jax.lax.Precision.HIGHEST) + fc_b)

    w_zh = Wzh_ref[...]
    w_rh = Wrh_ref[...]
    w_hh = Whh_ref[...]
    h = state
    c = jax.nn.relu(
        jnp.dot(state, Wch_ref[...], preferred_element_type=jnp.float32,
                precision=jax.lax.Precision.HIGHEST) + b_c)
    for t in range(TIME_WINDOW):
        xt = hist_ref[:, t:t + 1]
        z = jax.nn.sigmoid(
            jnp.dot(h, w_zh, preferred_element_type=jnp.float32,
                    precision=jax.lax.Precision.HIGHEST)
            + xt * w_zx + b_z + c)
        rr = jax.nn.sigmoid(
            jnp.dot(h, w_rh, preferred_element_type=jnp.float32,
                    precision=jax.lax.Precision.HIGHEST)
            + xt * w_rx + b_r + c)
        ht = jnp.tanh(
            jnp.dot(h * rr, w_hh, preferred_element_type=jnp.float32,
                    precision=jax.lax.Precision.HIGHEST)
            + xt * w_hx + b_h + c)
        h = (1.0 - z) * h + z * ht
    y = jnp.sum(h * pred_w, axis=1, keepdims=True) + pred_b
    out_ref[...] = jnp.broadcast_to(y, (b, 128))


def kernel(x, uu_sim, ii_sim, hist, user_embedding, item_embedding,
           user_cnn_w, user_cnn_b, item_cnn_w, item_cnn_b,
           int_W, int_b, fg_W, fg_b,
           W_zh, W_zx, b_z, W_rh, W_rx, b_r, W_hh, W_hx, b_h, W_ch, b_c,
           fc_W, fc_b, pred_W, pred_b):
    batch = x.shape[0]
    d = EMBED_DIM

    tk_u = _topk_indices(uu_sim)
    tk_i = _topk_indices(ii_sim)

    def _wexp(w):
        wf = jnp.zeros((16,), jnp.float32).at[:TOP_K].set(w)
        return jnp.broadcast_to(wf[:, None], (16, 16))

    u_idx = x[:, 1]
    i_idx = x[:, 2]
    emb_u128 = jnp.pad(user_embedding, ((0, 0), (0, 128 - d)))
    emb_i128 = jnp.pad(item_embedding, ((0, 0), (0, 128 - d)))
    self_u, nei_u, self_i, nei_i = _sc_combine(
        emb_u128, tk_u, u_idx, _wexp(user_cnn_w),
        emb_i128, tk_i, i_idx, _wexp(item_cnn_w))

    hist_t = hist.T
    wx = jnp.zeros((4, d), jnp.float32)
    wx = wx.at[0, :].set(W_zx[0])
    wx = wx.at[1, :].set(W_rx[0])
    wx = wx.at[2, :].set(W_hx[0])
    wx = wx.at[3, 0].set(pred_b[0])
    wx = wx.at[3, 1].set(user_cnn_b[0])
    wx = wx.at[3, 2].set(item_cnn_b[0])
    sp = jnp.stack([int_b, fg_b, b_z, b_r, b_h, b_c, fc_b, pred_W[:, 0]])

    full = lambda shape: pl.BlockSpec(shape, lambda bb: (0, 0))
    blk = lambda w: pl.BlockSpec((_BATCH_BLK, w), lambda bb: (bb, 0))
    out = pl.pallas_call(
        _batch_kernel,
        grid=(batch // _BATCH_BLK,),
        in_specs=[
            blk(128), blk(128), blk(128), blk(128),
            blk(TIME_WINDOW),
            full((d, d)), full((d, d)), full((d, d)), full((d, d)),
            full((d, d)), full((d, d)), full((4 * d, d)),
            full((4, d)), full((8, d)),
        ],
        out_specs=pl.BlockSpec((_BATCH_BLK, 128), lambda bb: (bb, 0)),
        out_shape=jax.ShapeDtypeStruct((batch, 128), jnp.float32),
        compiler_params=_PARALLEL,
    )(self_u, nei_u, self_i, nei_i, hist_t,
      int_W, fg_W, W_zh, W_rh, W_hh, W_ch, fc_W, wx, sp)
    return out[:, 0]


# speedup vs baseline: 1.8443x; 1.1470x over previous
import functools

import jax
import jax.numpy as jnp
from jax import lax
from jax.experimental import pallas as pl
from jax.experimental.pallas import tpu as pltpu
from jax.experimental.pallas import tpu_sc as plsc

NUM_TIMES = 64
EMBED_DIM = 64
TOP_K = 10
TIME_WINDOW = 8

_ROW_BLK = 256
_BATCH_BLK = 512

_PARALLEL = pltpu.CompilerParams(dimension_semantics=("parallel",))

_SC_NC = 2
_SC_NW = 32
_LANES = 16


def _topk_idx_kernel(sim_ref, out_ref):
    vals = sim_ref[...]
    r, n = vals.shape
    colf = lax.broadcasted_iota(jnp.int32, (r, n), 1).astype(jnp.float32)
    idxs = []
    for _ in range(TOP_K):
        m = jnp.max(vals, axis=1, keepdims=True)
        t = jnp.where(vals == m, colf, float(n))
        idxf = jnp.min(t, axis=1, keepdims=True)
        idxs.append(idxf.astype(jnp.int32))
        vals = jnp.where(t == idxf, -jnp.inf, vals)
    pad = jnp.broadcast_to(idxs[0], (r, 128 - TOP_K))
    out_ref[...] = jnp.concatenate(idxs + [pad], axis=1)


def _topk_indices(sim):
    n = sim.shape[0]
    return pl.pallas_call(
        _topk_idx_kernel,
        grid=(n // _ROW_BLK,),
        in_specs=[pl.BlockSpec((_ROW_BLK, n), lambda b: (b, 0))],
        out_specs=pl.BlockSpec((_ROW_BLK, 128), lambda b: (b, 0)),
        out_shape=jax.ShapeDtypeStruct((n, 128), jnp.int32),
        compiler_params=_PARALLEL,
    )(sim)


def _sc_combine(emb_u, tk_u, u_idx, w_u, emb_i, tk_i, i_idx, w_i):
    b = u_idx.shape[0]
    dd = emb_u.shape[1]
    bpw = b // _SC_NW
    sub = bpw // 2
    nj = EMBED_DIM // _LANES
    mesh = plsc.VectorSubcoreMesh(core_axis_name="c", subcore_axis_name="s")
    out_sd = jax.ShapeDtypeStruct((b, dd), jnp.float32)

    @functools.partial(
        pl.kernel,
        out_type=(out_sd, out_sd, out_sd, out_sd),
        mesh=mesh,
        compiler_params=pltpu.CompilerParams(needs_layout_passes=False),
        scratch_types=[
            pltpu.VMEM((sub,), jnp.int32),
            pltpu.VMEM((sub, 128), jnp.int32),
            pltpu.VMEM((16, sub), jnp.int32),
            pltpu.VMEM((sub, dd), jnp.float32),
            pltpu.VMEM((TOP_K, sub, dd), jnp.float32),
            pltpu.VMEM((sub, dd), jnp.float32),
            pltpu.VMEM((16, 16), jnp.float32),
            pltpu.SemaphoreType.DMA,
        ],
    )
    def combine_kernel(embu_hbm, tku_hbm, uid_hbm, wu_hbm,
                       embi_hbm, tki_hbm, iid_hbm, wi_hbm,
                       oselfu_hbm, oneiu_hbm, oselfi_hbm, oneii_hbm,
                       ids_v, tk_v, idxk_v, self_v, rows_v, nei_v, w_v, sem):
        wid = lax.axis_index("s") * _SC_NC + lax.axis_index("c")
        zero = jnp.zeros((_LANES,), jnp.float32)
        lane = lax.iota(jnp.int32, _LANES)

        def one_chunk(emb_hbm, tk_hbm, id_hbm, oself_hbm, onei_hbm, base):
            pltpu.sync_copy(id_hbm.at[pl.ds(base, sub)], ids_v)
            pltpu.async_copy(tk_hbm.at[ids_v], tk_v, sem).wait()
            pltpu.async_copy(emb_hbm.at[ids_v], self_v, sem).wait()
            pltpu.sync_copy(self_v, oself_hbm.at[pl.ds(base, sub)])
            copies = []
            for k in range(TOP_K):
                kcol = jnp.full((_LANES,), k, jnp.int32)
                for c in range(sub // _LANES):
                    rows16 = plsc.load_gather(
                        tk_v, [lane + _LANES * c, kcol])
                    idxk_v[k, pl.ds(_LANES * c, _LANES)] = rows16
                copies.append(
                    pltpu.async_copy(emb_hbm.at[idxk_v.at[k]],
                                     rows_v.at[k], sem))
            for cp in copies:
                cp.wait()

            def body(bb, _):
                for j in range(nj):
                    acc = zero
                    for k in range(TOP_K):
                        acc = acc + w_v[k] * rows_v[k, bb,
                                                    pl.ds(_LANES * j, _LANES)]
                    nei_v[bb, pl.ds(_LANES * j, _LANES)] = acc
                for j in range(nj, dd // _LANES):
                    nei_v[bb, pl.ds(_LANES * j, _LANES)] = zero
                return 0

            lax.fori_loop(0, sub, body, 0)
            pltpu.sync_copy(nei_v, onei_hbm.at[pl.ds(base, sub)])

        pltpu.sync_copy(wu_hbm, w_v)
        for h in range(2):
            one_chunk(embu_hbm, tku_hbm, uid_hbm, oselfu_hbm, oneiu_hbm,
                      wid * bpw + h * sub)
        pltpu.sync_copy(wi_hbm, w_v)
        for h in range(2):
            one_chunk(embi_hbm, tki_hbm, iid_hbm, oselfi_hbm, oneii_hbm,
                      wid * bpw + h * sub)

    return combine_kernel(emb_u, tk_u, u_idx, w_u, emb_i, tk_i, i_idx, w_i)


def _batch_kernel(ue_ref, un_ref, ie_ref, in_ref, hist_ref,
                  intW_ref, fgW_ref, Wzh_ref, Wrh_ref, Whh_ref, Wch_ref,
                  fcW_ref, wx_ref, sp_ref, out_ref):
    u_emb = ue_ref[:, :EMBED_DIM]
    i_emb = ie_ref[:, :EMBED_DIM]
    u_nei = un_ref[:, :EMBED_DIM] + wx_ref[3:4, 1:2]
    i_nei = in_ref[:, :EMBED_DIM] + wx_ref[3:4, 2:3]
    b = u_emb.shape[0]

    int_b = sp_ref[0:1, :]
    fg_b = sp_ref[1:2, :]
    b_z = sp_ref[2:3, :]
    b_r = sp_ref[3:4, :]
    b_h = sp_ref[4:5, :]
    b_c = sp_ref[5:6, :]
    fc_b = sp_ref[6:7, :]
    pred_w = sp_ref[7:8, :]
    w_zx = wx_ref[0:1, :]
    w_rx = wx_ref[1:2, :]
    w_hx = wx_ref[2:3, :]
    pred_b = wx_ref[3:4, 0:1]

    int_w = intW_ref[...]
    fg_w = fgW_ref[...]

    def interaction(a, bb):
        xv = a + bb
        fr = jax.nn.sigmoid(
            jnp.dot(xv, fg_w, preferred_element_type=jnp.float32,
                    precision=jax.lax.Precision.HIGHEST) + fg_b)
        t = jnp.dot(xv, int_w, preferred_element_type=jnp.float32,
                    precision=jax.lax.Precision.HIGHEST) + int_b
        return fr * t + (1.0 - fr) * xv

    cross = jnp.concatenate(
        [interaction(u_emb, u_nei), interaction(i_emb, i_nei), u_emb, i_emb],
        axis=1)
    state = jax.nn.relu(
        jnp.dot(cross, fcW_ref[...], preferred_element_type=jnp.float32,
                precision=jax.lax.Precision.HIGHEST) + fc_b)

    w_zh = Wzh_ref[...]
    w_rh = Wrh_ref[...]
    w_hh = Whh_ref[...]
    h = state
    c = jax.nn.relu(
        jnp.dot(state, Wch_ref[...], preferred_element_type=jnp.float32,
                precision=jax.lax.Precision.HIGHEST) + b_c)
    for t in range(TIME_WINDOW):
        xt = hist_ref[:, t:t + 1]
        z = jax.nn.sigmoid(
            jnp.dot(h, w_zh, preferred_element_type=jnp.float32,
                    precision=jax.lax.Precision.HIGHEST)
            + xt * w_zx + b_z + c)
        rr = jax.nn.sigmoid(
            jnp.dot(h, w_rh, preferred_element_type=jnp.float32,
                    precision=jax.lax.Precision.HIGHEST)
            + xt * w_rx + b_r + c)
        ht = jnp.tanh(
            jnp.dot(h * rr, w_hh, preferred_element_type=jnp.float32,
                    precision=jax.lax.Precision.HIGHEST)
            + xt * w_hx + b_h + c)
        h = (1.0 - z) * h + z * ht
    y = jnp.sum(h * pred_w, axis=1, keepdims=True) + pred_b
    out_ref[...] = jnp.broadcast_to(y, (b, 128))


def kernel(x, uu_sim, ii_sim, hist, user_embedding, item_embedding,
           user_cnn_w, user_cnn_b, item_cnn_w, item_cnn_b,
           int_W, int_b, fg_W, fg_b,
           W_zh, W_zx, b_z, W_rh, W_rx, b_r, W_hh, W_hx, b_h, W_ch, b_c,
           fc_W, fc_b, pred_W, pred_b):
    batch = x.shape[0]
    d = EMBED_DIM

    tk_u = _topk_indices(uu_sim)
    tk_i = _topk_indices(ii_sim)

    def _wexp(w):
        wf = jnp.zeros((16,), jnp.float32).at[:TOP_K].set(w)
        return jnp.broadcast_to(wf[:, None], (16, 16))

    u_idx = x[:, 1]
    i_idx = x[:, 2]
    emb_u128 = jnp.pad(user_embedding, ((0, 0), (0, 128 - d)))
    emb_i128 = jnp.pad(item_embedding, ((0, 0), (0, 128 - d)))
    self_u, nei_u, self_i, nei_i = _sc_combine(
        emb_u128, tk_u, u_idx, _wexp(user_cnn_w),
        emb_i128, tk_i, i_idx, _wexp(item_cnn_w))

    hist_t = hist.T
    wx = jnp.zeros((4, d), jnp.float32)
    wx = wx.at[0, :].set(W_zx[0])
    wx = wx.at[1, :].set(W_rx[0])
    wx = wx.at[2, :].set(W_hx[0])
    wx = wx.at[3, 0].set(pred_b[0])
    wx = wx.at[3, 1].set(user_cnn_b[0])
    wx = wx.at[3, 2].set(item_cnn_b[0])
    sp = jnp.stack([int_b, fg_b, b_z, b_r, b_h, b_c, fc_b, pred_W[:, 0]])

    full = lambda shape: pl.BlockSpec(shape, lambda bb: (0, 0))
    blk = lambda w: pl.BlockSpec((_BATCH_BLK, w), lambda bb: (bb, 0))
    out = pl.pallas_call(
        _batch_kernel,
        grid=(batch // _BATCH_BLK,),
        in_specs=[
            blk(128), blk(128), blk(128), blk(128),
            blk(TIME_WINDOW),
            full((d, d)), full((d, d)), full((d, d)), full((d, d)),
            full((d, d)), full((d, d)), full((4 * d, d)),
            full((4, d)), full((8, d)),
        ],
        out_specs=pl.BlockSpec((_BATCH_BLK, 128), lambda bb: (bb, 0)),
        out_shape=jax.ShapeDtypeStruct((batch, 128), jnp.float32),
        compiler_params=_PARALLEL,
    )(self_u, nei_u, self_i, nei_i, hist_t,
      int_W, fg_W, W_zh, W_rh, W_hh, W_ch, fc_W, wx, sp)
    return out[:, 0]


# speedup vs baseline: 1.9460x; 1.0551x over previous
import functools

import jax
import jax.numpy as jnp
from jax import lax
from jax.experimental import pallas as pl
from jax.experimental.pallas import tpu as pltpu
from jax.experimental.pallas import tpu_sc as plsc

NUM_TIMES = 64
EMBED_DIM = 64
TOP_K = 10
TIME_WINDOW = 8

_ROW_BLK = 256
_BATCH_BLK = 512

_PARALLEL = pltpu.CompilerParams(dimension_semantics=("parallel",))

_SC_NC = 2
_SC_NW = 32
_LANES = 16


def _topk_idx_kernel(sim_ref, out_ref):
    vals = sim_ref[...]
    r, n = vals.shape
    colf = lax.broadcasted_iota(jnp.int32, (r, n), 1).astype(jnp.float32)
    idxs = []
    for _ in range(TOP_K):
        m = jnp.max(vals, axis=1, keepdims=True)
        t = jnp.where(vals == m, colf, float(n))
        idxf = jnp.min(t, axis=1, keepdims=True)
        idxs.append(idxf.astype(jnp.int32))
        vals = jnp.where(t == idxf, -jnp.inf, vals)
    pad = jnp.broadcast_to(idxs[0], (r, 128 - TOP_K))
    out_ref[...] = jnp.concatenate(idxs + [pad], axis=1)


def _topk_indices(sim):
    n = sim.shape[0]
    return pl.pallas_call(
        _topk_idx_kernel,
        grid=(n // _ROW_BLK,),
        in_specs=[pl.BlockSpec((_ROW_BLK, n), lambda b: (b, 0))],
        out_specs=pl.BlockSpec((_ROW_BLK, 128), lambda b: (b, 0)),
        out_shape=jax.ShapeDtypeStruct((n, 128), jnp.int32),
        compiler_params=_PARALLEL,
    )(sim)


def _sc_combine(emb, tk, bidx, w):
    b = bidx.shape[0]
    dd = emb.shape[1]
    bpw = b // _SC_NW
    sub = bpw // 2
    nj = EMBED_DIM // _LANES
    mesh = plsc.VectorSubcoreMesh(core_axis_name="c", subcore_axis_name="s")
    out_sd = jax.ShapeDtypeStruct((b, dd), jnp.float32)

    @functools.partial(
        pl.kernel,
        out_type=(out_sd, out_sd),
        mesh=mesh,
        compiler_params=pltpu.CompilerParams(needs_layout_passes=False),
        scratch_types=[
            pltpu.VMEM((sub,), jnp.int32),
            pltpu.VMEM((sub, 128), jnp.int32),
            pltpu.VMEM((16, sub), jnp.int32),
            pltpu.VMEM((sub, dd), jnp.float32),
            pltpu.VMEM((TOP_K, sub, dd), jnp.float32),
            pltpu.VMEM((sub, dd), jnp.float32),
            pltpu.VMEM((16, 16), jnp.float32),
            pltpu.SemaphoreType.DMA,
        ],
    )
    def combine_kernel(emb_hbm, tk_hbm, id_hbm, w_hbm,
                       oself_hbm, onei_hbm,
                       ids_v, tk_v, idxk_v, self_v, rows_v, nei_v, w_v, sem):
        wid = lax.axis_index("s") * _SC_NC + lax.axis_index("c")
        zero = jnp.zeros((_LANES,), jnp.float32)
        lane = lax.iota(jnp.int32, _LANES)

        def one_chunk(base):
            pltpu.sync_copy(id_hbm.at[pl.ds(base, sub)], ids_v)
            pltpu.async_copy(tk_hbm.at[ids_v], tk_v, sem).wait()
            pltpu.async_copy(emb_hbm.at[ids_v], self_v, sem).wait()
            pltpu.sync_copy(self_v, oself_hbm.at[pl.ds(base, sub)])
            copies = []
            for k in range(TOP_K):
                kcol = jnp.full((_LANES,), k, jnp.int32)
                for c in range(sub // _LANES):
                    rows16 = plsc.load_gather(
                        tk_v, [lane + _LANES * c, kcol])
                    idxk_v[k, pl.ds(_LANES * c, _LANES)] = rows16
                copies.append(
                    pltpu.async_copy(emb_hbm.at[idxk_v.at[k]],
                                     rows_v.at[k], sem))
            for cp in copies:
                cp.wait()

            def body(bb, _):
                for j in range(nj):
                    acc = zero
                    for k in range(TOP_K):
                        acc = acc + w_v[k] * rows_v[k, bb,
                                                    pl.ds(_LANES * j, _LANES)]
                    nei_v[bb, pl.ds(_LANES * j, _LANES)] = acc
                for j in range(nj, dd // _LANES):
                    nei_v[bb, pl.ds(_LANES * j, _LANES)] = zero
                return 0

            lax.fori_loop(0, sub, body, 0)
            pltpu.sync_copy(nei_v, onei_hbm.at[pl.ds(base, sub)])

        pltpu.sync_copy(w_hbm, w_v)
        for h in range(2):
            one_chunk(wid * bpw + h * sub)

    return combine_kernel(emb, tk, bidx, w)


def _batch_kernel(ue_ref, un_ref, ie_ref, in_ref, hist_ref,
                  intW_ref, fgW_ref, Wzh_ref, Wrh_ref, Whh_ref, Wch_ref,
                  fcW_ref, wx_ref, sp_ref, out_ref):
    u_emb = ue_ref[:, :EMBED_DIM]
    i_emb = ie_ref[:, :EMBED_DIM]
    u_nei = un_ref[:, :EMBED_DIM] + wx_ref[3:4, 1:2]
    i_nei = in_ref[:, :EMBED_DIM] + wx_ref[3:4, 2:3]
    b = u_emb.shape[0]

    int_b = sp_ref[0:1, :]
    fg_b = sp_ref[1:2, :]
    b_z = sp_ref[2:3, :]
    b_r = sp_ref[3:4, :]
    b_h = sp_ref[4:5, :]
    b_c = sp_ref[5:6, :]
    fc_b = sp_ref[6:7, :]
    pred_w = sp_ref[7:8, :]
    w_zx = wx_ref[0:1, :]
    w_rx = wx_ref[1:2, :]
    w_hx = wx_ref[2:3, :]
    pred_b = wx_ref[3:4, 0:1]

    int_w = intW_ref[...]
    fg_w = fgW_ref[...]

    def interaction(a, bb):
        xv = a + bb
        fr = jax.nn.sigmoid(
            jnp.dot(xv, fg_w, preferred_element_type=jnp.float32,
                    precision=jax.lax.Precision.HIGHEST) + fg_b)
        t = jnp.dot(xv, int_w, preferred_element_type=jnp.float32,
                    precision=jax.lax.Precision.HIGHEST) + int_b
        return fr * t + (1.0 - fr) * xv

    cross = jnp.concatenate(
        [interaction(u_emb, u_nei), interaction(i_emb, i_nei), u_emb, i_emb],
        axis=1)
    state = jax.nn.relu(
        jnp.dot(cross, fcW_ref[...], preferred_element_type=jnp.float32,
                precision=jax.lax.Precision.HIGHEST) + fc_b)

    w_zh = Wzh_ref[...]
    w_rh = Wrh_ref[...]
    w_hh = Whh_ref[...]
    h = state
    c = jax.nn.relu(
        jnp.dot(state, Wch_ref[...], preferred_element_type=jnp.float32,
                precision=jax.lax.Precision.HIGHEST) + b_c)
    for t in range(TIME_WINDOW):
        xt = hist_ref[:, t:t + 1]
        z = jax.nn.sigmoid(
            jnp.dot(h, w_zh, preferred_element_type=jnp.float32,
                    precision=jax.lax.Precision.HIGHEST)
            + xt * w_zx + b_z + c)
        rr = jax.nn.sigmoid(
            jnp.dot(h, w_rh, preferred_element_type=jnp.float32,
                    precision=jax.lax.Precision.HIGHEST)
            + xt * w_rx + b_r + c)
        ht = jnp.tanh(
            jnp.dot(h * rr, w_hh, preferred_element_type=jnp.float32,
                    precision=jax.lax.Precision.HIGHEST)
            + xt * w_hx + b_h + c)
        h = (1.0 - z) * h + z * ht
    y = jnp.sum(h * pred_w, axis=1, keepdims=True) + pred_b
    out_ref[...] = jnp.broadcast_to(y, (b, 128))


def kernel(x, uu_sim, ii_sim, hist, user_embedding, item_embedding,
           user_cnn_w, user_cnn_b, item_cnn_w, item_cnn_b,
           int_W, int_b, fg_W, fg_b,
           W_zh, W_zx, b_z, W_rh, W_rx, b_r, W_hh, W_hx, b_h, W_ch, b_c,
           fc_W, fc_b, pred_W, pred_b):
    batch = x.shape[0]
    d = EMBED_DIM

    def _wexp(w):
        wf = jnp.zeros((16,), jnp.float32).at[:TOP_K].set(w)
        return jnp.broadcast_to(wf[:, None], (16, 16))

    u_idx = x[:, 1]
    i_idx = x[:, 2]
    emb_u128 = jnp.pad(user_embedding, ((0, 0), (0, 128 - d)))
    emb_i128 = jnp.pad(item_embedding, ((0, 0), (0, 128 - d)))

    tk_u = _topk_indices(uu_sim)
    self_u, nei_u = _sc_combine(emb_u128, tk_u, u_idx, _wexp(user_cnn_w))
    tk_i = _topk_indices(ii_sim)
    self_i, nei_i = _sc_combine(emb_i128, tk_i, i_idx, _wexp(item_cnn_w))

    hist_t = hist.T
    wx = jnp.zeros((4, d), jnp.float32)
    wx = wx.at[0, :].set(W_zx[0])
    wx = wx.at[1, :].set(W_rx[0])
    wx = wx.at[2, :].set(W_hx[0])
    wx = wx.at[3, 0].set(pred_b[0])
    wx = wx.at[3, 1].set(user_cnn_b[0])
    wx = wx.at[3, 2].set(item_cnn_b[0])
    sp = jnp.stack([int_b, fg_b, b_z, b_r, b_h, b_c, fc_b, pred_W[:, 0]])

    full = lambda shape: pl.BlockSpec(shape, lambda bb: (0, 0))
    blk = lambda w: pl.BlockSpec((_BATCH_BLK, w), lambda bb: (bb, 0))
    out = pl.pallas_call(
        _batch_kernel,
        grid=(batch // _BATCH_BLK,),
        in_specs=[
            blk(128), blk(128), blk(128), blk(128),
            blk(TIME_WINDOW),
            full((d, d)), full((d, d)), full((d, d)), full((d, d)),
            full((d, d)), full((d, d)), full((4 * d, d)),
            full((4, d)), full((8, d)),
        ],
        out_specs=pl.BlockSpec((_BATCH_BLK, 128), lambda bb: (bb, 0)),
        out_shape=jax.ShapeDtypeStruct((batch, 128), jnp.float32),
        compiler_params=_PARALLEL,
    )(self_u, nei_u, self_i, nei_i, hist_t,
      int_W, fg_W, W_zh, W_rh, W_hh, W_ch, fc_W, wx, sp)
    return out[:, 0]


# speedup vs baseline: 2.2601x; 1.1614x over previous
import functools

import jax
import jax.numpy as jnp
from jax import lax
from jax.experimental import pallas as pl
from jax.experimental.pallas import tpu as pltpu
from jax.experimental.pallas import tpu_sc as plsc

NUM_TIMES = 64
EMBED_DIM = 64
TOP_K = 10
TIME_WINDOW = 8

_ROW_BLK = 256
_BATCH_BLK = 512

_PARALLEL = pltpu.CompilerParams(dimension_semantics=("parallel",))

_SC_NC = 2
_SC_NW = 32
_LANES = 16


def _topk_idx_kernel(sim_ref, out_ref):
    vals = sim_ref[...]
    r, n = vals.shape
    colf = lax.broadcasted_iota(jnp.int32, (r, n), 1).astype(jnp.float32)
    idxs = []
    for _ in range(TOP_K):
        m = jnp.max(vals, axis=1, keepdims=True)
        t = jnp.where(vals == m, colf, float(n))
        idxf = jnp.min(t, axis=1, keepdims=True)
        idxs.append(idxf.astype(jnp.int32))
        vals = jnp.where(t == idxf, -jnp.inf, vals)
    pad = jnp.broadcast_to(idxs[0], (r, 128 - TOP_K))
    out_ref[...] = jnp.concatenate(idxs + [pad], axis=1)


def _topk_indices(sim):
    n = sim.shape[0]
    return pl.pallas_call(
        _topk_idx_kernel,
        grid=(n // _ROW_BLK,),
        in_specs=[pl.BlockSpec((_ROW_BLK, n), lambda b: (b, 0))],
        out_specs=pl.BlockSpec((_ROW_BLK, 128), lambda b: (b, 0)),
        out_shape=jax.ShapeDtypeStruct((n, 128), jnp.int32),
        compiler_params=_PARALLEL,
    )(sim)


def _sc_combine(emb, tk, bidx, w):
    b = bidx.shape[0]
    dd = emb.shape[1]
    bpw = b // _SC_NW
    sub = bpw // 2
    nj = EMBED_DIM // _LANES
    mesh = plsc.VectorSubcoreMesh(core_axis_name="c", subcore_axis_name="s")
    out_sd = jax.ShapeDtypeStruct((b, dd), jnp.float32)

    @functools.partial(
        pl.kernel,
        out_type=(out_sd, out_sd),
        mesh=mesh,
        compiler_params=pltpu.CompilerParams(needs_layout_passes=False),
        scratch_types=[
            pltpu.VMEM((sub,), jnp.int32),
            pltpu.VMEM((sub, 128), jnp.int32),
            pltpu.VMEM((16, sub), jnp.int32),
            pltpu.VMEM((sub, dd), jnp.float32),
            pltpu.VMEM((TOP_K, sub, dd), jnp.float32),
            pltpu.VMEM((sub, dd), jnp.float32),
            pltpu.VMEM((16, 16), jnp.float32),
            pltpu.SemaphoreType.DMA,
        ],
    )
    def combine_kernel(emb_hbm, tk_hbm, id_hbm, w_hbm,
                       oself_hbm, onei_hbm,
                       ids_v, tk_v, idxk_v, self_v, rows_v, nei_v, w_v, sem):
        wid = lax.axis_index("s") * _SC_NC + lax.axis_index("c")
        zero = jnp.zeros((_LANES,), jnp.float32)
        lane = lax.iota(jnp.int32, _LANES)

        def one_chunk(base):
            pltpu.sync_copy(id_hbm.at[pl.ds(base, sub)], ids_v)
            pltpu.async_copy(tk_hbm.at[ids_v], tk_v, sem).wait()
            pltpu.async_copy(emb_hbm.at[ids_v], self_v, sem).wait()
            pltpu.sync_copy(self_v, oself_hbm.at[pl.ds(base, sub)])
            copies = []
            for k in range(TOP_K):
                kcol = jnp.full((_LANES,), k, jnp.int32)
                for c in range(sub // _LANES):
                    rows16 = plsc.load_gather(
                        tk_v, [lane + _LANES * c, kcol])
                    idxk_v[k, pl.ds(_LANES * c, _LANES)] = rows16
                copies.append(
                    pltpu.async_copy(emb_hbm.at[idxk_v.at[k]],
                                     rows_v.at[k], sem))
            for cp in copies:
                cp.wait()

            def body(bb, _):
                for j in range(nj):
                    acc = zero
                    for k in range(TOP_K):
                        acc = acc + w_v[k] * rows_v[k, bb,
                                                    pl.ds(_LANES * j, _LANES)]
                    nei_v[bb, pl.ds(_LANES * j, _LANES)] = acc
                for j in range(nj, dd // _LANES):
                    nei_v[bb, pl.ds(_LANES * j, _LANES)] = zero
                return 0

            lax.fori_loop(0, sub, body, 0)
            pltpu.sync_copy(nei_v, onei_hbm.at[pl.ds(base, sub)])

        pltpu.sync_copy(w_hbm, w_v)
        for h in range(2):
            one_chunk(wid * bpw + h * sub)

    return combine_kernel(emb, tk, bidx, w)


def _batch_kernel(ue_ref, un_ref, ie_ref, in_ref, hist_ref,
                  fiW_ref, zrW_ref, Whh_ref, Wch_ref,
                  fcW_ref, wx_ref, sp_ref, out_ref):
    u_emb = ue_ref[:, :EMBED_DIM]
    i_emb = ie_ref[:, :EMBED_DIM]
    u_nei = un_ref[:, :EMBED_DIM] + wx_ref[3:4, 1:2]
    i_nei = in_ref[:, :EMBED_DIM] + wx_ref[3:4, 2:3]
    b = u_emb.shape[0]

    int_b = sp_ref[0:1, :]
    fg_b = sp_ref[1:2, :]
    b_z = sp_ref[2:3, :]
    b_r = sp_ref[3:4, :]
    b_h = sp_ref[4:5, :]
    b_c = sp_ref[5:6, :]
    fc_b = sp_ref[6:7, :]
    pred_w = sp_ref[7:8, :]
    w_zx = wx_ref[0:1, :]
    w_rx = wx_ref[1:2, :]
    w_hx = wx_ref[2:3, :]
    pred_b = wx_ref[3:4, 0:1]

    fi_w = fiW_ref[...]

    def interaction(a, bb):
        xv = a + bb
        ft = jnp.dot(xv, fi_w, preferred_element_type=jnp.float32)
        fr = jax.nn.sigmoid(ft[:, :EMBED_DIM] + fg_b)
        t = ft[:, EMBED_DIM:] + int_b
        return fr * t + (1.0 - fr) * xv

    cross = jnp.concatenate(
        [interaction(u_emb, u_nei), interaction(i_emb, i_nei), u_emb, i_emb],
        axis=1)
    state = jax.nn.relu(
        jnp.dot(cross, fcW_ref[...], preferred_element_type=jnp.float32) + fc_b)

    zr_w = zrW_ref[...]
    w_hh = Whh_ref[...]
    h = state
    c = jax.nn.relu(
        jnp.dot(state, Wch_ref[...], preferred_element_type=jnp.float32) + b_c)
    for t in range(TIME_WINDOW):
        xt = hist_ref[:, t:t + 1]
        zr = jnp.dot(h, zr_w, preferred_element_type=jnp.float32)
        z = jax.nn.sigmoid(zr[:, :EMBED_DIM] + xt * w_zx + b_z + c)
        rr = jax.nn.sigmoid(zr[:, EMBED_DIM:] + xt * w_rx + b_r + c)
        ht = jnp.tanh(
            jnp.dot(h * rr, w_hh, preferred_element_type=jnp.float32)
            + xt * w_hx + b_h + c)
        h = (1.0 - z) * h + z * ht
    y = jnp.sum(h * pred_w, axis=1, keepdims=True) + pred_b
    out_ref[...] = jnp.broadcast_to(y, (b, 128))


def kernel(x, uu_sim, ii_sim, hist, user_embedding, item_embedding,
           user_cnn_w, user_cnn_b, item_cnn_w, item_cnn_b,
           int_W, int_b, fg_W, fg_b,
           W_zh, W_zx, b_z, W_rh, W_rx, b_r, W_hh, W_hx, b_h, W_ch, b_c,
           fc_W, fc_b, pred_W, pred_b):
    batch = x.shape[0]
    d = EMBED_DIM

    def _wexp(w):
        wf = jnp.zeros((16,), jnp.float32).at[:TOP_K].set(w)
        return jnp.broadcast_to(wf[:, None], (16, 16))

    u_idx = x[:, 1]
    i_idx = x[:, 2]
    emb_u128 = jnp.pad(user_embedding, ((0, 0), (0, 128 - d)))
    emb_i128 = jnp.pad(item_embedding, ((0, 0), (0, 128 - d)))

    tk_u = _topk_indices(uu_sim)
    self_u, nei_u = _sc_combine(emb_u128, tk_u, u_idx, _wexp(user_cnn_w))
    tk_i = _topk_indices(ii_sim)
    self_i, nei_i = _sc_combine(emb_i128, tk_i, i_idx, _wexp(item_cnn_w))

    hist_t = hist.T
    wx = jnp.zeros((4, d), jnp.float32)
    wx = wx.at[0, :].set(W_zx[0])
    wx = wx.at[1, :].set(W_rx[0])
    wx = wx.at[2, :].set(W_hx[0])
    wx = wx.at[3, 0].set(pred_b[0])
    wx = wx.at[3, 1].set(user_cnn_b[0])
    wx = wx.at[3, 2].set(item_cnn_b[0])
    sp = jnp.stack([int_b, fg_b, b_z, b_r, b_h, b_c, fc_b, pred_W[:, 0]])
    fi_w = jnp.concatenate([fg_W, int_W], axis=1)
    zr_w = jnp.concatenate([W_zh, W_rh], axis=1)

    full = lambda shape: pl.BlockSpec(shape, lambda bb: (0, 0))
    blk = lambda w: pl.BlockSpec((_BATCH_BLK, w), lambda bb: (bb, 0))
    out = pl.pallas_call(
        _batch_kernel,
        grid=(batch // _BATCH_BLK,),
        in_specs=[
            blk(128), blk(128), blk(128), blk(128),
            blk(TIME_WINDOW),
            full((d, 2 * d)), full((d, 2 * d)),
            full((d, d)), full((d, d)), full((4 * d, d)),
            full((4, d)), full((8, d)),
        ],
        out_specs=pl.BlockSpec((_BATCH_BLK, 128), lambda bb: (bb, 0)),
        out_shape=jax.ShapeDtypeStruct((batch, 128), jnp.float32),
        compiler_params=_PARALLEL,
    )(self_u, nei_u, self_i, nei_i, hist_t,
      fi_w, zr_w, W_hh, W_ch, fc_W, wx, sp)
    return out[:, 0]
